# baseline (speedup 1.0000x reference)
import functools
import jax
import jax.numpy as jnp
from jax import lax
from jax.experimental import pallas as pl
from jax.experimental.pallas import tpu as pltpu
from jax.experimental.pallas import tpu_sc as plsc

WORDLEN = 100000
EMB = 64
CTX = 20
HID = 128

NC = 2
NS = 16
NW = NC * NS
SCCOLS = 99968
CW = 3200
TAILW = SCCOLS - (NW - 1) * CW
RB = 16
VCH = 25


def _stage1(x_ref, *refs):
    tbl = refs[:CTX]
    w1_ref, b1_ref = refs[CTX], refs[CTX + 1]
    hsel_out, idx_out, kk_out, h_out = refs[CTX + 2:CTX + 6]
    hs_smem, hsel_smem, idx_smem, kk_smem, csem = refs[CTX + 6:]

    acc = b1_ref[...]
    for i in range(CTX):
        blk = tbl[i][...]
        sub = lax.rem(x_ref[i], 8)
        msk = lax.broadcasted_iota(jnp.int32, (8, EMB), 0) == sub
        row = jnp.sum(jnp.where(msk, blk, 0.0), axis=0, keepdims=True)
        acc = acc + jnp.dot(row, w1_ref[i * EMB:(i + 1) * EMB, :],
                            preferred_element_type=jnp.float32)
    h_out[...] = jnp.maximum(acc, 0.0)

    pltpu.make_async_copy(h_out, hs_smem, csem).start()
    pltpu.make_async_copy(h_out, hs_smem, csem).wait()

    def zero_body(n, c):
        hsel_smem[0, n] = 0.0
        idx_smem[0, n] = 0
        return c

    lax.fori_loop(0, HID, zero_body, 0)

    def compact_body(n, cnt):
        v = hs_smem[0, n]
        nz = v > 0.0

        @pl.when(nz)
        def _():
            idx_smem[0, cnt] = n
            hsel_smem[0, cnt] = v

        return cnt + jnp.where(nz, 1, 0)

    k = lax.fori_loop(0, HID, compact_body, 0)

    def kfill(n, c):
        kk_smem[0, n] = k
        return c

    lax.fori_loop(0, 16, kfill, 0)

    pltpu.make_async_copy(hsel_smem, hsel_out, csem).start()
    pltpu.make_async_copy(hsel_smem, hsel_out, csem).wait()
    pltpu.make_async_copy(idx_smem, idx_out, csem).start()
    pltpu.make_async_copy(idx_smem, idx_out, csem).wait()
    pltpu.make_async_copy(kk_smem, kk_out, csem).start()
    pltpu.make_async_copy(kk_smem, kk_out, csem).wait()


def _run_stage1(x, table, W1, b1):
    b1r = b1.reshape(1, HID)
    tbl_specs = [
        pl.BlockSpec((8, EMB), lambda j, xr, i=i: (xr[i] // 8, 0))
        for i in range(CTX)
    ]
    grid_spec = pltpu.PrefetchScalarGridSpec(
        num_scalar_prefetch=1,
        grid=(1,),
        in_specs=[
            *tbl_specs,
            pl.BlockSpec((HID * 10, HID), lambda j, xr: (0, 0)),
            pl.BlockSpec((1, HID), lambda j, xr: (0, 0)),
        ],
        out_specs=[
            pl.BlockSpec((1, HID), lambda j, xr: (0, 0)),
            pl.BlockSpec((1, HID), lambda j, xr: (0, 0)),
            pl.BlockSpec((1, 16), lambda j, xr: (0, 0)),
            pl.BlockSpec((1, HID), lambda j, xr: (0, 0)),
        ],
        scratch_shapes=[
            pltpu.SMEM((1, HID), jnp.float32),
            pltpu.SMEM((1, HID), jnp.float32),
            pltpu.SMEM((1, HID), jnp.int32),
            pltpu.SMEM((1, 16), jnp.int32),
            pltpu.SemaphoreType.DMA,
        ],
    )
    return pl.pallas_call(
        _stage1,
        grid_spec=grid_spec,
        out_shape=(
            jax.ShapeDtypeStruct((1, HID), jnp.float32),
            jax.ShapeDtypeStruct((1, HID), jnp.int32),
            jax.ShapeDtypeStruct((1, 16), jnp.int32),
            jax.ShapeDtypeStruct((1, HID), jnp.float32),
        ),
    )(x, *([table] * CTX), W1, b1r)


_mesh = plsc.VectorSubcoreMesh(core_axis_name="c", subcore_axis_name="s")


@functools.partial(
    pl.kernel, mesh=_mesh,
    out_type=jax.ShapeDtypeStruct((SCCOLS,), jnp.float32),
    scratch_types=[
        pltpu.VMEM((HID,), jnp.int32),
        pltpu.VMEM((HID,), jnp.float32),
        pltpu.VMEM((16,), jnp.int32),
        pltpu.VMEM((2, RB, CW), jnp.float32),
        pltpu.VMEM((CW,), jnp.float32),
        pltpu.SemaphoreType.DMA,
        pltpu.SemaphoreType.DMA,
    ],
)
def _stage2(w2_hbm, idx_hbm, hsel_hbm, kk_hbm, out_hbm,
            idx_v, h_v, kk_v, buf, acc, sem0, sem1):
    wid = lax.axis_index("s") * NC + lax.axis_index("c")
    c0 = wid * CW
    is_tail = wid == NW - 1

    pltpu.sync_copy(idx_hbm, idx_v)
    pltpu.sync_copy(hsel_hbm, h_v)
    pltpu.sync_copy(kk_hbm, kk_v)
    kvec = kk_v[pl.ds(0, 16)]
    k = kvec[0]
    nb = (k + (RB - 1)) // RB

    def zero_chunks(nv):
        z = jnp.zeros((16,), jnp.float32)
        for v in range(nv):
            acc[pl.ds(v * 16, 16)] = z

    @pl.when(jnp.logical_not(is_tail))
    def _():
        zero_chunks(CW // 16)

    @pl.when(is_tail)
    def _():
        zero_chunks(TAILW // 16)

    def gather(bb, pb, width):
        return pltpu.async_copy(
            w2_hbm.at[idx_v.at[pl.ds(bb * RB, RB)], pl.ds(c0, width)],
            buf.at[pb, pl.ds(0, RB), pl.ds(0, width)],
            sem0,
        )

    def issue(bb, width):
        gather(bb, lax.rem(bb, 2), width).start()

    def mac_batch(bb, pb, nch, vch):
        hv = h_v[pl.ds(bb * RB, RB)]

        def chunk_body(cc, c):
            base = cc * (vch * 16)
            for r in range(RB):
                hr = jnp.broadcast_to(hv[r], (16,))
                for v in range(vch):
                    off = base + v * 16
                    plsc.addupdate(
                        acc.at[pl.ds(off, 16)],
                        hr * buf[pb, r, pl.ds(off, 16)])
            return c

        lax.fori_loop(0, nch, chunk_body, 0)

    def run(width, nch, vch):
        @pl.when(nb > 0)
        def _():
            issue(0, width)

        def body(bb, c):
            pb = lax.rem(bb, 2)

            @pl.when(bb + 1 < nb)
            def _():
                issue(bb + 1, width)

            gather(bb, pb, width).wait()
            mac_batch(bb, pb, nch, vch)
            return c

        lax.fori_loop(0, nb, body, 0)
        pltpu.sync_copy(acc.at[pl.ds(0, width)],
                        out_hbm.at[pl.ds(c0, width)])

    @pl.when(jnp.logical_not(is_tail))
    def _():
        run(CW, CW // (VCH * 16), VCH)

    @pl.when(is_tail)
    def _():
        run(TAILW, TAILW // 256, 16)


def _stage3(lg_ref, h_ref, w2t_ref, b2_ref, b2t_ref, out_ref):
    lo = lg_ref[...] + b2_ref[...]
    lt = jnp.dot(h_ref[...], w2t_ref[...],
                 preferred_element_type=jnp.float32) + b2t_ref[...]
    mx = jnp.maximum(jnp.max(lo), jnp.max(lt))
    s = jnp.sum(jnp.exp(lo - mx)) + jnp.sum(jnp.exp(lt - mx))
    lse = mx + jnp.log(s)
    out_ref[:, pl.ds(0, SCCOLS)] = lo - lse
    out_ref[:, SCCOLS:WORDLEN] = lt - lse


def _run_stage3(logits0, h, W2, b2):
    lgr = logits0.reshape(1, SCCOLS)
    w2t = W2[:, SCCOLS:]
    b2r = b2.reshape(1, WORDLEN)
    b2t = b2r[:, SCCOLS:]
    return pl.pallas_call(
        _stage3,
        grid=(1,),
        in_specs=[
            pl.BlockSpec((1, SCCOLS), lambda j: (0, 0)),
            pl.BlockSpec((1, HID), lambda j: (0, 0)),
            pl.BlockSpec((HID, WORDLEN - SCCOLS), lambda j: (0, 0)),
            pl.BlockSpec((1, SCCOLS), lambda j: (0, 0)),
            pl.BlockSpec((1, WORDLEN - SCCOLS), lambda j: (0, 0)),
        ],
        out_specs=pl.BlockSpec((1, WORDLEN), lambda j: (0, 0)),
        out_shape=jax.ShapeDtypeStruct((1, WORDLEN), jnp.float32),
    )(lgr, h, w2t, b2r, b2t)


def kernel(x, table, W1, b1, W2, b2):
    hsel, idxp, kk, h = _run_stage1(x, table, W1, b1)
    logits0 = _stage2(W2, idxp.reshape(HID), hsel.reshape(HID),
                      kk.reshape(16))
    return _run_stage3(logits0, h, W2, b2)


# speedup vs baseline: 1.8131x; 1.8131x over previous
import jax
import jax.numpy as jnp
from jax import lax
from jax.experimental import pallas as pl
from jax.experimental.pallas import tpu as pltpu

WORDLEN = 100000
EMB = 64
CTX = 20
HID = 128
BK = 2048
G = 7
NJ = 7
PAD = G * NJ * BK
NEG = -jnp.inf


def _fused(x_ref, *refs):
    tbl = refs[:CTX]
    w1_ref, b1_ref = refs[CTX], refs[CTX + 1]
    w2_blks = refs[CTX + 2:CTX + 2 + G]
    b2_ref, out_ref, h_ref, m_ref = refs[CTX + 2 + G:]
    j = pl.program_id(0)

    @pl.when(j == 0)
    def _compute_h():
        acc = b1_ref[...]
        for i in range(CTX):
            blk = tbl[i][...]
            sub = lax.rem(x_ref[i], 8)
            msk = lax.broadcasted_iota(jnp.int32, (8, EMB), 0) == sub
            row = jnp.sum(jnp.where(msk, blk, 0.0), axis=0, keepdims=True)
            acc = acc + jnp.dot(row, w1_ref[i * EMB:(i + 1) * EMB, :],
                                preferred_element_type=jnp.float32)
        h_ref[...] = jnp.maximum(acc, 0.0)
        m_ref[...] = jnp.full((1, BK), NEG, jnp.float32)

    h = h_ref[...]
    m = m_ref[...]
    for g in range(G):
        bidx = g * NJ + j
        logits = jnp.dot(h, w2_blks[g][...],
                         preferred_element_type=jnp.float32)
        logits = logits + b2_ref[:, pl.ds(bidx * BK, BK)]
        col = lax.broadcasted_iota(jnp.int32, (1, BK), 1) + bidx * BK
        logits = jnp.where(col < WORDLEN, logits, NEG)
        out_ref[:, pl.ds(bidx * BK, BK)] = logits
        m = jnp.maximum(m, logits)
    m_ref[...] = m

    @pl.when(j == NJ - 1)
    def _finalize():
        mx = jnp.max(m_ref[...])
        lo = out_ref[...]
        s = jnp.sum(jnp.exp(lo - mx))
        out_ref[...] = lo - (mx + jnp.log(s))


def kernel(x, table, W1, b1, W2, b2):
    b1r = b1.reshape(1, HID)
    b2p = jnp.pad(b2, (0, PAD - WORDLEN)).reshape(1, PAD)

    tbl_specs = [
        pl.BlockSpec((8, EMB), lambda j, xr, i=i: (xr[i] // 8, 0))
        for i in range(CTX)
    ]
    w2_specs = [
        pl.BlockSpec((HID, BK), lambda j, xr, g=g: (0, g * NJ + j))
        for g in range(G)
    ]
    grid_spec = pltpu.PrefetchScalarGridSpec(
        num_scalar_prefetch=1,
        grid=(NJ,),
        in_specs=[
            *tbl_specs,
            pl.BlockSpec((HID * 10, HID), lambda j, xr: (0, 0)),
            pl.BlockSpec((1, HID), lambda j, xr: (0, 0)),
            *w2_specs,
            pl.BlockSpec((1, PAD), lambda j, xr: (0, 0)),
        ],
        out_specs=pl.BlockSpec((1, PAD), lambda j, xr: (0, 0)),
        scratch_shapes=[
            pltpu.VMEM((1, HID), jnp.float32),
            pltpu.VMEM((1, BK), jnp.float32),
        ],
    )

    out = pl.pallas_call(
        _fused,
        grid_spec=grid_spec,
        out_shape=jax.ShapeDtypeStruct((1, PAD), jnp.float32),
    )(x, *([table] * CTX), W1, b1r, *([W2] * G), b2p)
    return out[:, :WORDLEN]
